# Initial kernel scaffold; baseline (speedup 1.0000x reference)
import jax
import jax.numpy as jnp
from jax.experimental import pallas as pl


def kernel(x, edge_index, W1, a_s1, a_d1, b1, W2, a_s2, a_d2, b2, W3, a_s3, a_d3, b3):
    raise NotImplementedError("write your pallas kernel here")


# speedup vs baseline: 19.9155x; 19.9155x over previous
import functools

import jax
import jax.numpy as jnp
from jax import lax
from jax.experimental import pallas as pl
from jax.experimental.pallas import tpu as pltpu
from jax.experimental.pallas import tpu_sc as plsc

N = 10000
D = 128
NP = 10112
E_RAW = 320000
E_TOT = E_RAW + N
NC = 2
NS = 16
NWORK = NC * NS
BB = 128
BPW = -(-E_TOT // (NWORK * BB))
E_PAD = NWORK * BPW * BB
EPW = BPW * BB
RPS = NP // NS
RB = 632
F32 = jnp.float32


def _tc_first_body(x_ref, w_ref, asw_ref, adw_ref, h_ref, as_ref, ad_ref):
    h = jnp.dot(x_ref[...], w_ref[...], preferred_element_type=F32)
    h_ref[...] = h
    as_ref[...] = jnp.sum(h * asw_ref[...], axis=-1, keepdims=True)
    ad_ref[...] = jnp.sum(h * adw_ref[...], axis=-1, keepdims=True)


def _tc_mid_body(ph_ref, pd_ref, b_ref, w_ref, asw_ref, adw_ref,
                 h_ref, as_ref, ad_ref):
    agg = ph_ref[0] + ph_ref[1]
    den = pd_ref[0, :, 0:1] + pd_ref[1, :, 0:1]
    xl = agg / (den + 1e-16) + b_ref[...]
    xl = jnp.where(xl > 0, xl, jnp.exp(jnp.minimum(xl, 0.0)) - 1.0)
    h = jnp.dot(xl, w_ref[...], preferred_element_type=F32)
    h_ref[...] = h
    as_ref[...] = jnp.sum(h * asw_ref[...], axis=-1, keepdims=True)
    ad_ref[...] = jnp.sum(h * adw_ref[...], axis=-1, keepdims=True)


def _tc_final_body(ph_ref, pd_ref, b_ref, out_ref):
    agg = ph_ref[0] + ph_ref[1]
    den = pd_ref[0, :, 0:1] + pd_ref[1, :, 0:1]
    out_ref[...] = agg / (den + 1e-16) + b_ref[...]


_SPEC_W = pl.BlockSpec((D, D), lambda i: (0, 0))
_SPEC_AW = pl.BlockSpec((1, D), lambda i: (0, 0))
_SPEC_ROWS = pl.BlockSpec((RB, D), lambda i: (i, 0))
_SPEC_COL = pl.BlockSpec((RB, 1), lambda i: (i, 0))
_SPEC_PH = pl.BlockSpec((NC, RB, D), lambda i: (0, i, 0))
_SPEC_PD = pl.BlockSpec((NC, RB, 16), lambda i: (0, i, 0))

_OUT_HAA = [jax.ShapeDtypeStruct((NP, D), F32),
            jax.ShapeDtypeStruct((NP, 1), F32),
            jax.ShapeDtypeStruct((NP, 1), F32)]


def _tc_first(xp, w, asw, adw):
    return pl.pallas_call(
        _tc_first_body,
        grid=(NP // RB,),
        in_specs=[_SPEC_ROWS, _SPEC_W, _SPEC_AW, _SPEC_AW],
        out_specs=[_SPEC_ROWS, _SPEC_COL, _SPEC_COL],
        out_shape=_OUT_HAA,
    )(xp, w, asw, adw)


def _tc_mid(ph, pdn, b, w, asw, adw):
    return pl.pallas_call(
        _tc_mid_body,
        grid=(NP // RB,),
        in_specs=[_SPEC_PH, _SPEC_PD, _SPEC_AW, _SPEC_W, _SPEC_AW, _SPEC_AW],
        out_specs=[_SPEC_ROWS, _SPEC_COL, _SPEC_COL],
        out_shape=_OUT_HAA,
    )(ph, pdn, b, w, asw, adw)


def _tc_final(ph, pdn, b):
    return pl.pallas_call(
        _tc_final_body,
        grid=(NP // RB,),
        in_specs=[_SPEC_PH, _SPEC_PD, _SPEC_AW],
        out_specs=_SPEC_ROWS,
        out_shape=jax.ShapeDtypeStruct((NP, D), F32),
    )(ph, pdn, b)


def _sc_edge_body(h_hbm, as_hbm, ad_hbm, src_hbm, dst_hbm,
                  ph_hbm, pd_hbm,
                  as_v, ad_v, src_v, dst_v, p_v, rows_v, pcol_v,
                  acc_h, acc_d, sem):
    cid = lax.axis_index("c")
    sid = lax.axis_index("s")
    wid = sid * NC + cid

    pltpu.sync_copy(as_hbm, as_v)
    pltpu.sync_copy(ad_hbm, ad_v)

    def _zero_row(i, carry):
        for j in range(D // 16):
            rows_v[i, pl.ds(j * 16, 16)] = jnp.zeros((16,), F32)
        pcol_v[i, :] = jnp.zeros((16,), F32)
        return carry

    lax.fori_loop(0, BB, _zero_row, 0)
    for c in range(-(-RPS // BB)):
        r0 = sid * RPS + c * BB
        nr = min(BB, RPS - c * BB)
        pltpu.sync_copy(rows_v.at[pl.ds(0, nr)], acc_h.at[pl.ds(r0, nr)])
        pltpu.sync_copy(pcol_v.at[pl.ds(0, nr)], acc_d.at[pl.ds(r0, nr)])
    plsc.subcore_barrier()

    def _block(b, carry):
        base = wid * EPW + b * BB
        pltpu.sync_copy(src_hbm.at[pl.ds(base, BB)], src_v)
        pltpu.sync_copy(dst_hbm.at[pl.ds(base, BB)], dst_v)
        for k in range(BB // 16):
            s_idx = src_v[pl.ds(k * 16, 16)]
            d_idx = dst_v[pl.ds(k * 16, 16)]
            e = plsc.load_gather(as_v, [s_idx]) + plsc.load_gather(ad_v, [d_idx])
            e = jnp.where(e > 0, e, 0.2 * e)
            p_v[pl.ds(k * 16, 16)] = jnp.exp(e)
        pltpu.async_copy(h_hbm.at[src_v], rows_v, sem).wait()

        def _row(i, c2):
            pb = plsc.load_gather(p_v, [jnp.full((16,), i, jnp.int32)])
            for j in range(D // 16):
                rows_v[i, pl.ds(j * 16, 16)] = rows_v[i, pl.ds(j * 16, 16)] * pb
            pcol_v[i, :] = pb
            return c2

        lax.fori_loop(0, BB, _row, 0)
        pltpu.sync_copy(rows_v, acc_h.at[dst_v], add=True)
        pltpu.sync_copy(pcol_v, acc_d.at[dst_v], add=True)
        return carry

    lax.fori_loop(0, BPW, _block, 0)
    plsc.subcore_barrier()

    for c in range(-(-RPS // BB)):
        r0 = sid * RPS + c * BB
        nr = min(BB, RPS - c * BB)
        pltpu.sync_copy(acc_h.at[pl.ds(r0, nr)], ph_hbm.at[cid, pl.ds(r0, nr)])
        pltpu.sync_copy(acc_d.at[pl.ds(r0, nr)], pd_hbm.at[cid, pl.ds(r0, nr)])


_sc_edge = functools.partial(
    pl.kernel,
    mesh=plsc.VectorSubcoreMesh(core_axis_name="c", subcore_axis_name="s"),
    compiler_params=pltpu.CompilerParams(needs_layout_passes=False,
                                         use_tc_tiling_on_sc=False),
    out_type=[jax.ShapeDtypeStruct((NC, NP, D), F32),
              jax.ShapeDtypeStruct((NC, NP, 16), F32)],
    scratch_types=[
        pltpu.VMEM((NP,), F32),
        pltpu.VMEM((NP,), F32),
        pltpu.VMEM((BB,), jnp.int32),
        pltpu.VMEM((BB,), jnp.int32),
        pltpu.VMEM((BB,), F32),
        pltpu.VMEM((BB, D), F32),
        pltpu.VMEM((BB, 16), F32),
        pltpu.VMEM_SHARED((NP, D), F32),
        pltpu.VMEM_SHARED((NP, 16), F32),
        pltpu.SemaphoreType.DMA,
    ],
)(_sc_edge_body)


def kernel(x, edge_index, W1, a_s1, a_d1, b1, W2, a_s2, a_d2, b2,
           W3, a_s3, a_d3, b3):
    pad_e = E_PAD - E_TOT
    loops = jnp.arange(N, dtype=jnp.int32)
    src = jnp.concatenate([edge_index[0], loops,
                           jnp.zeros((pad_e,), jnp.int32)])
    dst = jnp.concatenate([edge_index[1], loops,
                           jnp.full((pad_e,), N, jnp.int32)])
    xp = jnp.pad(x, ((0, NP - N), (0, 0)))

    h, asf, adf = _tc_first(xp, W1, a_s1, a_d1)
    ph, pdn = _sc_edge(h, asf.reshape(NP), adf.reshape(NP), src, dst)
    h, asf, adf = _tc_mid(ph, pdn, b1.reshape(1, D), W2, a_s2, a_d2)
    ph, pdn = _sc_edge(h, asf.reshape(NP), adf.reshape(NP), src, dst)
    h, asf, adf = _tc_mid(ph, pdn, b2.reshape(1, D), W3, a_s3, a_d3)
    ph, pdn = _sc_edge(h, asf.reshape(NP), adf.reshape(NP), src, dst)
    out = _tc_final(ph, pdn, b3.reshape(1, D))
    return out[:N]
